# Initial kernel scaffold; baseline (speedup 1.0000x reference)
import jax
import jax.numpy as jnp
from jax.experimental import pallas as pl


def kernel(values, bin_values):
    raise NotImplementedError("write your pallas kernel here")


# speedup vs baseline: 32.8272x; 32.8272x over previous
import functools

import jax
import jax.numpy as jnp
from jax.experimental import pallas as pl
from jax.experimental.pallas import tpu as pltpu


def _twohot_body(v_ref, b_ref, o_ref, *, rows, nbins):
    bins = b_ref[0:1, :]
    v = v_ref[...]
    v = jnp.clip(v, b_ref[0, 0], b_ref[0, nbins - 1])
    lt = bins < v
    idx = jnp.sum(lt.astype(jnp.float32), axis=1, keepdims=True)
    lv = jnp.max(jnp.where(lt, bins, -jnp.inf), axis=1, keepdims=True)
    rv = jnp.min(jnp.where(lt, jnp.inf, bins), axis=1, keepdims=True)
    dist = jnp.maximum(rv - lv, 1e-8)
    rw = (v - lv) / dist
    p = jnp.where(idx < 0.5, 0.0, (idx - 1.0) + rw)
    j = jax.lax.broadcasted_iota(jnp.int32, (rows, nbins), 1).astype(jnp.float32)
    o_ref[...] = jnp.maximum(0.0, 1.0 - jnp.abs(p - j))


def kernel(values, bin_values):
    orig_shape = values.shape
    n = values.size
    nbins = bin_values.shape[0]
    rows = 1024
    assert n % rows == 0
    grid = n // rows
    flat = values.reshape(n, 1)
    bins2d = bin_values.reshape(1, nbins)
    out = pl.pallas_call(
        functools.partial(_twohot_body, rows=rows, nbins=nbins),
        grid=(grid,),
        in_specs=[
            pl.BlockSpec((rows, 1), lambda i: (i, 0)),
            pl.BlockSpec((1, nbins), lambda i: (0, 0)),
        ],
        out_specs=pl.BlockSpec((rows, nbins), lambda i: (i, 0)),
        out_shape=jax.ShapeDtypeStruct((n, nbins), jnp.float32),
        compiler_params=pltpu.CompilerParams(
            dimension_semantics=("arbitrary",),
        ),
    )(flat, bins2d)
    return out.reshape(*orig_shape, nbins)


# speedup vs baseline: 68.4399x; 2.0849x over previous
import functools

import jax
import jax.numpy as jnp
from jax.experimental import pallas as pl
from jax.experimental.pallas import tpu as pltpu


def _twohot_body(v_ref, b_ref, u1_ref, au1_ref, u2_ref, au2_ref, o_ref):
    v = jnp.maximum(v_ref[...], b_ref[0, 0, 0])
    t1 = jnp.clip(v * u1_ref[...] - au1_ref[...], 0.0, 1.0)
    t2 = jnp.clip(v * u2_ref[...] - au2_ref[...], 0.0, 1.0)
    o_ref[...] = t1 - t2


def kernel(values, bin_values):
    r0, r1 = values.shape
    nbins = bin_values.shape[0]
    bins = bin_values
    u1i = 1.0 / (bins[1:] - bins[:-1])
    u1 = jnp.concatenate([jnp.zeros((1,), jnp.float32), u1i])
    au1 = jnp.concatenate([jnp.full((1,), -1.0, jnp.float32), bins[:-1] * u1i])
    nxt = jnp.concatenate([bins[1:], bins[-1:]])
    d2 = nxt - bins
    u2 = jnp.where(d2 > 0, 1.0 / jnp.maximum(d2, 1e-30), 0.0)
    au2 = bins * u2

    B = 128
    assert r0 % B == 0
    grid = r0 // B
    v3 = values.reshape(r0, r1, 1)

    def c3(x):
        return x.reshape(1, 1, nbins)

    cspec = pl.BlockSpec((1, 1, nbins), lambda i: (0, 0, 0))
    out = pl.pallas_call(
        _twohot_body,
        grid=(grid,),
        in_specs=[
            pl.BlockSpec((B, r1, 1), lambda i: (i, 0, 0)),
            cspec, cspec, cspec, cspec, cspec,
        ],
        out_specs=pl.BlockSpec((B, r1, nbins), lambda i: (i, 0, 0)),
        out_shape=jax.ShapeDtypeStruct((r0, r1, nbins), jnp.float32),
        compiler_params=pltpu.CompilerParams(
            dimension_semantics=("arbitrary",),
        ),
    )(v3, c3(bins), c3(u1), c3(au1), c3(u2), c3(au2))
    return out


# speedup vs baseline: 69.2904x; 1.0124x over previous
import functools

import jax
import jax.numpy as jnp
from jax.experimental import pallas as pl
from jax.experimental.pallas import tpu as pltpu


def _twohot_body(v_ref, b_ref, u1_ref, au1_ref, u2_ref, au2_ref, o_ref):
    v = jnp.maximum(v_ref[...], b_ref[0, 0, 0])
    t1 = jnp.clip(v * u1_ref[...] - au1_ref[...], 0.0, 1.0)
    t2 = jnp.clip(v * u2_ref[...] - au2_ref[...], 0.0, 1.0)
    o_ref[...] = t1 - t2


def kernel(values, bin_values):
    r0, r1 = values.shape
    nbins = bin_values.shape[0]
    bins = bin_values
    u1i = 1.0 / (bins[1:] - bins[:-1])
    u1 = jnp.concatenate([jnp.zeros((1,), jnp.float32), u1i])
    au1 = jnp.concatenate([jnp.full((1,), -1.0, jnp.float32), bins[:-1] * u1i])
    nxt = jnp.concatenate([bins[1:], bins[-1:]])
    d2 = nxt - bins
    u2 = jnp.where(d2 > 0, 1.0 / jnp.maximum(d2, 1e-30), 0.0)
    au2 = bins * u2

    B = 256
    assert r0 % B == 0
    grid = r0 // B
    v3 = values.reshape(r0, r1, 1)

    def c3(x):
        return x.reshape(1, 1, nbins)

    cspec = pl.BlockSpec((1, 1, nbins), lambda i: (0, 0, 0))
    out = pl.pallas_call(
        _twohot_body,
        grid=(grid,),
        in_specs=[
            pl.BlockSpec((B, r1, 1), lambda i: (i, 0, 0)),
            cspec, cspec, cspec, cspec, cspec,
        ],
        out_specs=pl.BlockSpec((B, r1, nbins), lambda i: (i, 0, 0)),
        out_shape=jax.ShapeDtypeStruct((r0, r1, nbins), jnp.float32),
        compiler_params=pltpu.CompilerParams(
            dimension_semantics=("arbitrary",),
        ),
    )(v3, c3(bins), c3(u1), c3(au1), c3(u2), c3(au2))
    return out
